# baseline (speedup 1.0000x reference)
import functools

import jax
import jax.numpy as jnp
from jax import lax
from jax.experimental import pallas as pl
from jax.experimental.pallas import tpu as pltpu
from jax.experimental.pallas import tpu_sc as plsc

_NUM_CORES = 2
_NUM_SUBCORES = 16
_NUM_WORKERS = _NUM_CORES * _NUM_SUBCORES

_CHUNK = 256
_DP = 128
_RB = 128

_mesh = plsc.VectorSubcoreMesh(
    core_axis_name="c",
    subcore_axis_name="s",
    num_cores=_NUM_CORES,
    num_subcores=_NUM_SUBCORES,
)


def _relayout_call(xt, n_rows, d):
    n_full = n_rows // _RB
    per_w = n_full // _NUM_WORKERS
    n_extra = n_full - per_w * _NUM_WORKERS
    tail = n_rows - n_full * _RB

    @functools.partial(
        pl.kernel,
        out_type=jax.ShapeDtypeStruct((n_rows, _DP), jnp.float32),
        mesh=_mesh,
        compiler_params=pltpu.CompilerParams(
            use_tc_tiling_on_sc=True, needs_layout_passes=False),
        scratch_types=[
            pltpu.VMEM((d, _RB), jnp.float32),
            pltpu.VMEM((d, _RB), jnp.float32),
            pltpu.VMEM((_RB, _DP), jnp.float32),
            pltpu.VMEM((_RB, _DP), jnp.float32),
            pltpu.VMEM((d, _RB // 2), jnp.float32),
            pltpu.SemaphoreType.DMA,
            pltpu.SemaphoreType.DMA,
            pltpu.SemaphoreType.DMA,
            pltpu.SemaphoreType.DMA,
        ],
    )
    def relayout_kernel(xt_hbm, xr_hbm, in0, in1, ob0, ob1, tailbuf,
                        isem0, isem1, osem0, osem1):
        wid = lax.axis_index("s") * _NUM_CORES + lax.axis_index("c")
        in_b = (in0, in1)
        ob_b = (ob0, ob1)
        isem_b = (isem0, isem1)
        osem_b = (osem0, osem1)
        lvecs = [
            lax.broadcasted_iota(jnp.int32, (16,), 0) + 16 * k
            for k in range(_RB // 16)
        ]

        def block_row0(i):
            return (i * _NUM_WORKERS + wid) * _RB

        def start_in(i, slot):
            pltpu.async_copy(
                xt_hbm.at[:, pl.ds(block_row0(i), _RB)], in_b[slot],
                isem_b[slot])

        def transpose_block(vbuf, obuf):
            @pl.loop(0, d)
            def _col(c):
                cvec = jnp.zeros((16,), jnp.int32) + c
                for k in range(_RB // 16):
                    plsc.store_scatter(
                        obuf, [lvecs[k], cvec], vbuf[c, pl.ds(16 * k, 16)])

        start_in(0, 0)

        @pl.loop(0, per_w)
        def _ring(i):
            slot = lax.rem(i, 2)
            for s in range(2):
                @pl.when(slot == s)
                def _():
                    pltpu.make_async_copy(
                        xt_hbm.at[:, pl.ds(block_row0(i), _RB)], in_b[s],
                        isem_b[s]).wait()

                    @pl.when(i + 1 < per_w)
                    def _():
                        start_in(i + 1, 1 - s)

                    @pl.when(i >= 2)
                    def _():
                        pltpu.make_async_copy(
                            ob_b[s],
                            xr_hbm.at[pl.ds(block_row0(i - 2), _RB)],
                            osem_b[s]).wait()
                    transpose_block(in_b[s], ob_b[s])
                    pltpu.async_copy(
                        ob_b[s], xr_hbm.at[pl.ds(block_row0(i), _RB)],
                        osem_b[s])

        for j in range(max(0, per_w - 2), per_w):
            s = j % 2
            pltpu.make_async_copy(
                ob_b[s], xr_hbm.at[pl.ds(block_row0(j), _RB)],
                osem_b[s]).wait()

        @pl.when(wid < n_extra)
        def _():
            r0 = (per_w * _NUM_WORKERS + wid) * _RB
            pltpu.sync_copy(xt_hbm.at[:, pl.ds(r0, _RB)], in0)
            transpose_block(in0, ob0)
            pltpu.sync_copy(ob0, xr_hbm.at[pl.ds(r0, _RB)])

        if tail:
            @pl.when(wid == _NUM_WORKERS - 1)
            def _():
                r0 = n_full * _RB
                pltpu.sync_copy(
                    xt_hbm.at[:, pl.ds(r0, tail)], tailbuf.at[:, pl.ds(0, tail)])

                @pl.loop(0, d)
                def _col(c):
                    cvec = jnp.zeros((16,), jnp.int32) + c
                    for k in range(tail // 16):
                        plsc.store_scatter(
                            ob1, [lvecs[k], cvec],
                            tailbuf[c, pl.ds(16 * k, 16)])
                pltpu.sync_copy(
                    ob1.at[pl.ds(0, tail)], xr_hbm.at[pl.ds(r0, tail)])

    return relayout_kernel(xt)


def _gather(xr, index, b):
    rows_per_worker = b // _NUM_WORKERS
    nch = rows_per_worker // _CHUNK

    @functools.partial(
        pl.kernel,
        out_type=jax.ShapeDtypeStruct((b, _DP), jnp.float32),
        mesh=_mesh,
        compiler_params=pltpu.CompilerParams(use_tc_tiling_on_sc=True),
        scratch_types=[
            pltpu.VMEM((_CHUNK,), jnp.int32),
            pltpu.VMEM((_CHUNK,), jnp.int32),
            pltpu.VMEM((_CHUNK, _DP), jnp.float32),
            pltpu.VMEM((_CHUNK, _DP), jnp.float32),
            pltpu.SemaphoreType.DMA,
            pltpu.SemaphoreType.DMA,
            pltpu.SemaphoreType.DMA,
            pltpu.SemaphoreType.DMA,
        ],
    )
    def gather_kernel(x_hbm, idx_hbm, out_hbm, idx0, idx1, rows0, rows1,
                      gsem0, gsem1, osem0, osem1):
        wid = lax.axis_index("s") * _NUM_CORES + lax.axis_index("c")
        base = wid * rows_per_worker
        idx_b = (idx0, idx1)
        rows_b = (rows0, rows1)
        gsem_b = (gsem0, gsem1)
        osem_b = (osem0, osem1)

        def load_and_gather(g, slot):
            off = base + g * _CHUNK
            pltpu.sync_copy(idx_hbm.at[pl.ds(off, _CHUNK)], idx_b[slot])
            pltpu.async_copy(x_hbm.at[idx_b[slot]], rows_b[slot], gsem_b[slot])

        load_and_gather(0, 0)
        load_and_gather(1, 1)

        @pl.loop(0, nch // 2 - 1)
        def _ring(h):
            g = 2 * h
            for slot in range(2):
                gcur = g + slot
                pltpu.make_async_copy(
                    x_hbm.at[idx_b[slot]], rows_b[slot], gsem_b[slot]).wait()
                pltpu.async_copy(
                    rows_b[slot],
                    out_hbm.at[pl.ds(base + gcur * _CHUNK, _CHUNK)],
                    osem_b[slot])
                pltpu.make_async_copy(
                    rows_b[slot],
                    out_hbm.at[pl.ds(base + gcur * _CHUNK, _CHUNK)],
                    osem_b[slot]).wait()
                load_and_gather(gcur + 2, slot)

        for slot in range(2):
            gcur = nch - 2 + slot
            pltpu.make_async_copy(
                x_hbm.at[idx_b[slot]], rows_b[slot], gsem_b[slot]).wait()
            pltpu.sync_copy(
                rows_b[slot], out_hbm.at[pl.ds(base + gcur * _CHUNK, _CHUNK)])

    return gather_kernel(xr, index)


@functools.partial(jax.jit, static_argnums=(2, 3))
def _gather_call(x, index, b, d):
    xr = _relayout_call(x.T, x.shape[0], d)
    return _gather(xr, index, b)


def kernel(x, dim, index):
    del dim
    b = index.shape[0]
    d = x.shape[1]
    return _gather_call(x, index, b, d)[:, :d]


# speedup vs baseline: 1.3291x; 1.3291x over previous
import functools

import jax
import jax.numpy as jnp
from jax import lax
from jax.experimental import pallas as pl
from jax.experimental.pallas import tpu as pltpu
from jax.experimental.pallas import tpu_sc as plsc

_NUM_CORES = 2
_NUM_SUBCORES = 16
_NUM_WORKERS = _NUM_CORES * _NUM_SUBCORES

_CHUNK = 256
_DP = 128


@functools.partial(jax.jit, static_argnums=(2,))
def _gather_call(x, index, b):
    rows_per_worker = b // _NUM_WORKERS
    nch = rows_per_worker // _CHUNK
    xp = jnp.zeros((x.shape[0], _DP), jnp.float32).at[:, : x.shape[1]].set(x)
    mesh = plsc.VectorSubcoreMesh(
        core_axis_name="c",
        subcore_axis_name="s",
        num_cores=_NUM_CORES,
        num_subcores=_NUM_SUBCORES,
    )

    @functools.partial(
        pl.kernel,
        out_type=jax.ShapeDtypeStruct((b, _DP), jnp.float32),
        mesh=mesh,
        compiler_params=pltpu.CompilerParams(use_tc_tiling_on_sc=True),
        scratch_types=[
            pltpu.VMEM((_CHUNK,), jnp.int32),
            pltpu.VMEM((_CHUNK,), jnp.int32),
            pltpu.VMEM((_CHUNK, _DP), jnp.float32),
            pltpu.VMEM((_CHUNK, _DP), jnp.float32),
            pltpu.SemaphoreType.DMA,
            pltpu.SemaphoreType.DMA,
            pltpu.SemaphoreType.DMA,
            pltpu.SemaphoreType.DMA,
        ],
    )
    def gather_kernel(x_hbm, idx_hbm, out_hbm, idx0, idx1, rows0, rows1,
                      gsem0, gsem1, osem0, osem1):
        wid = lax.axis_index("s") * _NUM_CORES + lax.axis_index("c")
        base = wid * rows_per_worker
        idx_b = (idx0, idx1)
        rows_b = (rows0, rows1)
        gsem_b = (gsem0, gsem1)
        osem_b = (osem0, osem1)

        def load_and_gather(g, slot):
            off = base + g * _CHUNK
            pltpu.sync_copy(idx_hbm.at[pl.ds(off, _CHUNK)], idx_b[slot])
            pltpu.async_copy(x_hbm.at[idx_b[slot]], rows_b[slot], gsem_b[slot])

        load_and_gather(0, 0)
        load_and_gather(1, 1)

        @pl.loop(0, nch // 2 - 1)
        def _ring(h):
            g = 2 * h
            for slot in range(2):
                gcur = g + slot
                pltpu.make_async_copy(
                    x_hbm.at[idx_b[slot]], rows_b[slot], gsem_b[slot]).wait()
                pltpu.async_copy(
                    rows_b[slot],
                    out_hbm.at[pl.ds(base + gcur * _CHUNK, _CHUNK)],
                    osem_b[slot])
                pltpu.make_async_copy(
                    rows_b[slot],
                    out_hbm.at[pl.ds(base + gcur * _CHUNK, _CHUNK)],
                    osem_b[slot]).wait()
                load_and_gather(gcur + 2, slot)

        for slot in range(2):
            gcur = nch - 2 + slot
            pltpu.make_async_copy(
                x_hbm.at[idx_b[slot]], rows_b[slot], gsem_b[slot]).wait()
            pltpu.sync_copy(
                rows_b[slot], out_hbm.at[pl.ds(base + gcur * _CHUNK, _CHUNK)])

    return gather_kernel(xp, index)


def kernel(x, dim, index):
    del dim
    b = index.shape[0]
    d = x.shape[1]
    return _gather_call(x, index, b)[:, :d]


# speedup vs baseline: 1.8652x; 1.4034x over previous
import functools

import jax
import jax.numpy as jnp
from jax import lax
from jax.experimental import pallas as pl
from jax.experimental.pallas import tpu as pltpu
from jax.experimental.pallas import tpu_sc as plsc

_NUM_CORES = 2
_NUM_SUBCORES = 16
_NUM_WORKERS = _NUM_CORES * _NUM_SUBCORES

_CHUNK = 416
_DP = 128


@functools.partial(jax.jit, static_argnums=(2,))
def _gather_call(x, index, b):
    rows_per_worker = b // _NUM_WORKERS
    nch = rows_per_worker // _CHUNK
    xp = jnp.pad(x, ((0, 0), (0, _DP - x.shape[1])))
    mesh = plsc.VectorSubcoreMesh(
        core_axis_name="c",
        subcore_axis_name="s",
        num_cores=_NUM_CORES,
        num_subcores=_NUM_SUBCORES,
    )

    @functools.partial(
        pl.kernel,
        out_type=jax.ShapeDtypeStruct((b, _DP), jnp.float32),
        mesh=mesh,
        compiler_params=pltpu.CompilerParams(use_tc_tiling_on_sc=True),
        scratch_types=[
            pltpu.VMEM((_CHUNK,), jnp.int32),
            pltpu.VMEM((_CHUNK,), jnp.int32),
            pltpu.VMEM((_CHUNK, _DP), jnp.float32),
            pltpu.VMEM((_CHUNK, _DP), jnp.float32),
            pltpu.SemaphoreType.DMA,
            pltpu.SemaphoreType.DMA,
            pltpu.SemaphoreType.DMA,
            pltpu.SemaphoreType.DMA,
        ],
    )
    def gather_kernel(x_hbm, idx_hbm, out_hbm, idx0, idx1, rows0, rows1,
                      gsem0, gsem1, osem0, osem1):
        wid = lax.axis_index("s") * _NUM_CORES + lax.axis_index("c")
        base = wid * rows_per_worker
        idx_b = (idx0, idx1)
        rows_b = (rows0, rows1)
        gsem_b = (gsem0, gsem1)
        osem_b = (osem0, osem1)

        def load_and_gather(g, slot):
            off = base + g * _CHUNK
            pltpu.sync_copy(idx_hbm.at[pl.ds(off, _CHUNK)], idx_b[slot])
            pltpu.async_copy(x_hbm.at[idx_b[slot]], rows_b[slot], gsem_b[slot])

        load_and_gather(0, 0)
        load_and_gather(1, 1)

        @pl.loop(0, nch // 2 - 1)
        def _ring(h):
            g = 2 * h
            for slot in range(2):
                gcur = g + slot
                pltpu.make_async_copy(
                    x_hbm.at[idx_b[slot]], rows_b[slot], gsem_b[slot]).wait()
                pltpu.async_copy(
                    rows_b[slot],
                    out_hbm.at[pl.ds(base + gcur * _CHUNK, _CHUNK)],
                    osem_b[slot])
                pltpu.make_async_copy(
                    rows_b[slot],
                    out_hbm.at[pl.ds(base + gcur * _CHUNK, _CHUNK)],
                    osem_b[slot]).wait()
                load_and_gather(gcur + 2, slot)

        for slot in range(2):
            gcur = nch - 2 + slot
            pltpu.make_async_copy(
                x_hbm.at[idx_b[slot]], rows_b[slot], gsem_b[slot]).wait()
            pltpu.sync_copy(
                rows_b[slot], out_hbm.at[pl.ds(base + gcur * _CHUNK, _CHUNK)])

    return gather_kernel(xp, index)


def kernel(x, dim, index):
    del dim
    b = index.shape[0]
    d = x.shape[1]
    return _gather_call(x, index, b)[:, :d]
